# Initial kernel scaffold; baseline (speedup 1.0000x reference)
import jax
import jax.numpy as jnp
from jax.experimental import pallas as pl


def kernel(x, pos, charge, edge_index, edge_attr, W1, b1, ln1_g, ln1_b, W2, b2, nW1, nb1, nln_g, nln_b, cW1, cb1, cW2, cb2):
    raise NotImplementedError("write your pallas kernel here")


# speedup vs baseline: 3.4098x; 3.4098x over previous
import math

import jax
import jax.numpy as jnp
from jax import lax
from jax.experimental import pallas as pl
from jax.experimental.pallas import tpu as pltpu
from jax.experimental.pallas import tpu_sc as plsc

N = 10000
E = 320000
H = 128
NR = 16
CUTOFF = 5.0

NC = 2
NS = 16
NW = NC * NS
GC = 128
NCHUNK = E // GC
NPAD = 10240
NPT = NPAD // NS

MBLK = 400
EBLK = 512


def _silu(v):
    return v * jax.nn.sigmoid(v)


def _nodepre_body(x_ref, w_ref, b_ref, p_ref, q_ref):
    r = jnp.dot(x_ref[...], w_ref[...], preferred_element_type=jnp.float32)
    p_ref[...] = r[:, 0:128] + b_ref[...]
    q_ref[...] = r[:, 128:256]


def _nodepre(x, w1ab, b1r):
    return pl.pallas_call(
        _nodepre_body,
        grid=(N // MBLK,),
        in_specs=[
            pl.BlockSpec((MBLK, H), lambda i: (i, 0)),
            pl.BlockSpec((H, 2 * H), lambda i: (0, 0)),
            pl.BlockSpec((1, H), lambda i: (0, 0)),
        ],
        out_specs=[
            pl.BlockSpec((MBLK, H), lambda i: (i, 0)),
            pl.BlockSpec((MBLK, H), lambda i: (i, 0)),
        ],
        out_shape=[
            jax.ShapeDtypeStruct((N, H), jnp.float32),
            jax.ShapeDtypeStruct((N, H), jnp.float32),
        ],
    )(x, w1ab, b1r)


def _gather_body(pp, qq, srcv, dstv, stab_hbm, pre_out, efs_out,
                 sidx, didx, pbuf, qbuf, sobuf, stab, sem1, sem2):
    c = lax.axis_index("c")
    s = lax.axis_index("s")
    wid = s * NC + c
    nj = jnp.where(wid < NCHUNK % NW, NCHUNK // NW + 1, NCHUNK // NW)

    pltpu.sync_copy(stab_hbm, stab)

    def chunk(j, carry):
        off = (wid + j * NW) * GC
        pltpu.sync_copy(srcv.at[pl.ds(off, GC)], sidx)
        pltpu.sync_copy(dstv.at[pl.ds(off, GC)], didx)
        cp1 = pltpu.async_copy(pp.at[sidx], pbuf, sem1)
        cp2 = pltpu.async_copy(qq.at[didx], qbuf, sem2)
        cp1.wait()
        cp2.wait()

        def row(i, carry2):
            for k in range(8):
                sl = pl.ds(k * 16, 16)
                pbuf[i, sl] = pbuf[i, sl] + qbuf[i, sl]
            return carry2

        lax.fori_loop(0, GC, row, 0)

        for g in range(GC // 16):
            gsl = pl.ds(g * 16, 16)
            rs = sidx[gsl] * 4
            rd = didx[gsl] * 4
            for k in range(4):
                sobuf[k, gsl] = plsc.load_gather(stab, [rs + k])
                sobuf[4 + k, gsl] = plsc.load_gather(stab, [rd + k])

        pltpu.sync_copy(pbuf, pre_out.at[pl.ds(off, GC)])
        pltpu.sync_copy(sobuf, efs_out.at[:, pl.ds(off, GC)])
        return carry

    lax.fori_loop(0, nj, chunk, 0)


def _gather(p, q, src, dst, stab1d):
    mesh = plsc.VectorSubcoreMesh(core_axis_name="c", subcore_axis_name="s")
    return pl.kernel(
        _gather_body,
        out_type=[
            jax.ShapeDtypeStruct((E, H), jnp.float32),
            jax.ShapeDtypeStruct((8, E), jnp.float32),
        ],
        mesh=mesh,
        scratch_types=[
            pltpu.VMEM((GC,), jnp.int32),
            pltpu.VMEM((GC,), jnp.int32),
            pltpu.VMEM((GC, H), jnp.float32),
            pltpu.VMEM((GC, H), jnp.float32),
            pltpu.VMEM((8, GC), jnp.float32),
            pltpu.VMEM((4 * N,), jnp.float32),
            pltpu.SemaphoreType.DMA,
            pltpu.SemaphoreType.DMA,
        ],
        compiler_params=pltpu.CompilerParams(needs_layout_passes=False),
    )(p, q, src, dst, stab1d)


def _edge_body(pre_ref, efs_ref, ea_ref, w1cr_ref, w1cx_ref, w2_ref, cw1_ref,
               vec_ref, h_ref, st_ref):
    nrow = pre_ref.shape[0]
    i8a = lax.broadcasted_iota(jnp.int32, (8, 8), 0)
    i8b = lax.broadcasted_iota(jnp.int32, (8, 8), 1)
    eye8 = (i8a == i8b).astype(jnp.float32)

    s8 = lax.dot_general(efs_ref[...], eye8, (((0,), (0,)), ((), ())),
                         preferred_element_type=jnp.float32)
    ps = s8[:, 0:3]
    cs = s8[:, 3:4]
    pd = s8[:, 4:7]
    cd = s8[:, 7:8]
    ea = ea_ref[...]

    rel = pd - ps
    dist = jnp.sqrt(jnp.sum(rel * rel, axis=-1, keepdims=True) + 1e-8)
    clp = jnp.maximum(dist, 1e-6)
    unit = rel / clp

    def leg_mean_abs(a):
        co = jnp.cos(a)
        p2 = (3.0 * co * co - 1.0) * 0.5
        p3 = (5.0 * co * p2 - 2.0 * co) / 3.0
        return (1.0 + jnp.abs(co) + jnp.abs(p2) + jnp.abs(p3)) * 0.25

    a_s = leg_mean_abs(ea[:, 0:1]) * ea[:, 2:3]
    d_s = leg_mean_abs(ea[:, 1:2]) * ea[:, 3:4]
    gate = jnp.clip(1.0 + 0.6 * (a_s + d_s), 0.35, 2.5)

    freq = (lax.broadcasted_iota(jnp.int32, (1, NR), 1).astype(jnp.float32)
            + 1.0) * (math.pi / CUTOFF)
    radial = jnp.sin(freq * clp) / clp

    sp = jnp.dot(radial * gate, w1cr_ref[...],
                 preferred_element_type=jnp.float32)
    sp = sp + (dist * (gate / CUTOFF)) * w1cx_ref[0:1, :]
    sp = sp + (cs * cd * gate) * w1cx_ref[1:2, :]
    sp = sp + (jnp.abs(cs - cd) * gate) * w1cx_ref[2:3, :]

    h = _silu(pre_ref[...] + sp)
    m = jnp.mean(h, axis=-1, keepdims=True)
    hc = h - m
    var = jnp.mean(hc * hc, axis=-1, keepdims=True)
    hn = hc * lax.rsqrt(var + 1e-5) * vec_ref[0:1, :] + vec_ref[1:2, :]
    h2 = _silu(jnp.dot(hn, w2_ref[...], preferred_element_type=jnp.float32)
               + vec_ref[2:3, :])

    t = _silu(jnp.dot(h2, cw1_ref[...], preferred_element_type=jnp.float32)
              + vec_ref[3:4, 0:64])
    coord = jnp.sum(t * vec_ref[4:5, 0:64], axis=-1, keepdims=True) \
        + vec_ref[5:6, 0:1]
    ucg = unit * (coord * gate)

    h_ref[...] = h2
    m8 = jnp.concatenate([ucg, jnp.zeros((nrow, 5), jnp.float32)], axis=1)
    st_ref[...] = lax.dot_general(eye8, m8, (((1,), (1,)), ((), ())),
                                  preferred_element_type=jnp.float32)


def _edge_mlp(pre, efs, edge_attr, w1cr, w1cx, w2, cw1, vecc):
    return pl.pallas_call(
        _edge_body,
        grid=(E // EBLK,),
        in_specs=[
            pl.BlockSpec((EBLK, H), lambda i: (i, 0)),
            pl.BlockSpec((8, EBLK), lambda i: (0, i)),
            pl.BlockSpec((EBLK, 4), lambda i: (i, 0)),
            pl.BlockSpec((NR, H), lambda i: (0, 0)),
            pl.BlockSpec((8, H), lambda i: (0, 0)),
            pl.BlockSpec((H, H), lambda i: (0, 0)),
            pl.BlockSpec((H, 64), lambda i: (0, 0)),
            pl.BlockSpec((8, H), lambda i: (0, 0)),
        ],
        out_specs=[
            pl.BlockSpec((EBLK, H), lambda i: (i, 0)),
            pl.BlockSpec((8, EBLK), lambda i: (0, i)),
        ],
        out_shape=[
            jax.ShapeDtypeStruct((E, H), jnp.float32),
            jax.ShapeDtypeStruct((8, E), jnp.float32),
        ],
    )(pre, efs, edge_attr, w1cr, w1cx, w2, cw1, vecc)


HALF = NPAD // 2
ZPT = HALF // NS


def _scatter_body(hpay, spay, srcv, zrow, agg_out, smallp_out,
                  hbuf, spbuf, idxv, idx2, acc, slab, sem):
    c = lax.axis_index("c")
    s = lax.axis_index("s")
    nj = jnp.where(s < NCHUNK % NS, NCHUNK // NS + 1, NCHUNK // NS)

    pltpu.sync_copy(zrow, slab.at[pl.ds(s * ZPT, ZPT)])

    @pl.when(c == 0)
    def _():
        def zloop(i, carry):
            acc[pl.ds(i * 16, 16)] = jnp.zeros((16,), jnp.float32)
            return carry

        lax.fori_loop(0, (4 * N) // 16, zloop, 0)

    plsc.subcore_barrier()

    ones16 = jnp.ones((16,), jnp.float32)
    base_node = c * HALF

    def chunk(j, carry):
        off = (s + j * NS) * GC
        pltpu.sync_copy(srcv.at[pl.ds(off, GC)], idxv)
        pltpu.sync_copy(hpay.at[pl.ds(off, GC)], hbuf)
        for g in range(GC // 16):
            gsl = pl.ds(g * 16, 16)
            r = idxv[gsl] - base_node
            ok = (r >= 0) & (r < HALF)
            idx2[gsl] = jnp.where(ok, r, HALF)
        pltpu.sync_copy(hbuf, slab.at[idx2], add=True)

        @pl.when(c == 0)
        def _():
            pltpu.sync_copy(spay.at[:, pl.ds(off, GC)], spbuf)
            for g in range(GC // 16):
                gsl = pl.ds(g * 16, 16)
                rows = idxv[gsl]
                for k in range(3):
                    plsc.addupdate_scatter(acc, [rows + k * N],
                                           spbuf[k, gsl])
                plsc.addupdate_scatter(acc, [rows + 3 * N], ones16)

        return carry

    lax.fori_loop(0, nj, chunk, 0)
    plsc.subcore_barrier()
    pltpu.sync_copy(slab.at[pl.ds(s * ZPT, ZPT)],
                    agg_out.at[pl.ds(c * HALF + s * ZPT, ZPT)])

    @pl.when(c == 0)
    def _():
        pltpu.sync_copy(acc, smallp_out.at[s])


def _scatter(hpay, spay, src, zrow):
    mesh = plsc.VectorSubcoreMesh(core_axis_name="c", subcore_axis_name="s")
    return pl.kernel(
        _scatter_body,
        out_type=[
            jax.ShapeDtypeStruct((NPAD, H), jnp.float32),
            jax.ShapeDtypeStruct((NS, 4 * N), jnp.float32),
        ],
        mesh=mesh,
        scratch_types=[
            pltpu.VMEM((GC, H), jnp.float32),
            pltpu.VMEM((8, GC), jnp.float32),
            pltpu.VMEM((GC,), jnp.int32),
            pltpu.VMEM((GC,), jnp.int32),
            pltpu.VMEM((4 * N,), jnp.float32),
            pltpu.VMEM_SHARED((HALF + 8, H), jnp.float32),
            pltpu.SemaphoreType.DMA,
        ],
        compiler_params=pltpu.CompilerParams(needs_layout_passes=False),
    )(hpay, spay, src, zrow)


def _final_body(x_ref, a_ref, sp_ref, pos_ref, nwa_ref, nwb_ref, vec_ref,
                xo_ref, po_ref):
    nrow = x_ref.shape[0]
    i4a = lax.broadcasted_iota(jnp.int32, (4, 4), 0)
    i4b = lax.broadcasted_iota(jnp.int32, (4, 4), 1)
    eye4 = (i4a == i4b).astype(jnp.float32)

    s4 = jnp.sum(sp_ref[...], axis=2)
    s4t = lax.dot_general(s4, eye4, (((0,), (0,)), ((), ())),
                          preferred_element_type=jnp.float32)
    cnt = jnp.maximum(s4t[:, 3:4], 1.0)
    delta = s4t[:, 0:3] / cnt
    agg = a_ref[...] / cnt

    xv = x_ref[...]
    t = _silu(jnp.dot(xv, nwa_ref[...], preferred_element_type=jnp.float32)
              + jnp.dot(agg, nwb_ref[...], preferred_element_type=jnp.float32)
              + vec_ref[0:1, :])
    m = jnp.mean(t, axis=-1, keepdims=True)
    tc = t - m
    var = jnp.mean(tc * tc, axis=-1, keepdims=True)
    tn = tc * lax.rsqrt(var + 1e-5) * vec_ref[1:2, :] + vec_ref[2:3, :]
    xo_ref[...] = xv + tn
    po_ref[:, 0:3] = pos_ref[...] + 0.1 * delta
    po_ref[:, 3:8] = jnp.zeros((nrow, 5), jnp.float32)


def _final(x, aggp, smallp3, pos, nw1a, nw1b, vece):
    return pl.pallas_call(
        _final_body,
        grid=(N // MBLK,),
        in_specs=[
            pl.BlockSpec((MBLK, H), lambda i: (i, 0)),
            pl.BlockSpec((MBLK, H), lambda i: (i, 0)),
            pl.BlockSpec((4, MBLK, NS), lambda i: (0, i, 0)),
            pl.BlockSpec((MBLK, 3), lambda i: (i, 0)),
            pl.BlockSpec((H, H), lambda i: (0, 0)),
            pl.BlockSpec((H, H), lambda i: (0, 0)),
            pl.BlockSpec((8, H), lambda i: (0, 0)),
        ],
        out_specs=[
            pl.BlockSpec((MBLK, H), lambda i: (i, 0)),
            pl.BlockSpec((MBLK, 8), lambda i: (i, 0)),
        ],
        out_shape=[
            jax.ShapeDtypeStruct((N, H), jnp.float32),
            jax.ShapeDtypeStruct((N, 8), jnp.float32),
        ],
    )(x, aggp, smallp3, pos, nw1a, nw1b, vece)


def kernel(x, pos, charge, edge_index, edge_attr, W1, b1, ln1_g, ln1_b, W2,
           b2, nW1, nb1, nln_g, nln_b, cW1, cb1, cW2, cb2):
    f32 = jnp.float32
    src = edge_index[0].astype(jnp.int32)
    dst = edge_index[1].astype(jnp.int32)

    w1ab = jnp.concatenate([W1[0:H], W1[H:2 * H]], axis=1)
    b1r = b1.reshape(1, H)
    p, q = _nodepre(x, w1ab, b1r)

    stab1d = jnp.concatenate([pos, charge], axis=1).reshape(-1)
    pre, efs = _gather(p, q, src, dst, stab1d)

    w1c = W1[2 * H:]
    w1cr = w1c[0:NR]
    w1cx = jnp.concatenate([w1c[NR:], jnp.zeros((5, H), f32)], axis=0)
    vecc = jnp.stack([
        ln1_g, ln1_b, b2,
        jnp.pad(cb1, (0, H - 64)),
        jnp.pad(cW2[:, 0], (0, H - 64)),
        jnp.pad(cb2, (0, H - 1)),
        jnp.zeros((H,), f32), jnp.zeros((H,), f32),
    ])
    hpay, spay = _edge_mlp(pre, efs, edge_attr, w1cr, w1cx, W2, cW1, vecc)

    zrow = jnp.zeros((ZPT, H), f32)
    aggs, smallp = _scatter(hpay, spay, src, zrow)

    vece = jnp.stack([nb1, nln_g, nln_b] + [jnp.zeros((H,), f32)] * 5)
    xo, po = _final(x, aggs,
                    smallp.reshape(NS, 4, N).transpose(1, 2, 0), pos,
                    nW1[0:H], nW1[H:2 * H], vece)
    return (xo, po[:, 0:3])


# speedup vs baseline: 5.7875x; 1.6973x over previous
import math

import jax
import jax.numpy as jnp
from jax import lax
from jax.experimental import pallas as pl
from jax.experimental.pallas import tpu as pltpu
from jax.experimental.pallas import tpu_sc as plsc

N = 10000
E = 320000
H = 128
NR = 16
CUTOFF = 5.0

NC = 2
NS = 16
NW = NC * NS
GC = 128
NCHUNK = E // GC
NPAD = 10240
NPT = NPAD // NS

MBLK = 400
FBLK = 512
EBLK = 512


def _silu(v):
    return v * jax.nn.sigmoid(v)


def _nodepre_body(x_ref, w_ref, b_ref, p_ref, q_ref):
    r = jnp.dot(x_ref[...], w_ref[...], preferred_element_type=jnp.float32)
    p_ref[...] = r[:, 0:128] + b_ref[...]
    q_ref[...] = r[:, 128:256]


def _nodepre(x, w1ab, b1r):
    return pl.pallas_call(
        _nodepre_body,
        grid=(N // MBLK,),
        in_specs=[
            pl.BlockSpec((MBLK, H), lambda i: (i, 0)),
            pl.BlockSpec((H, 2 * H), lambda i: (0, 0)),
            pl.BlockSpec((1, H), lambda i: (0, 0)),
        ],
        out_specs=[
            pl.BlockSpec((MBLK, H), lambda i: (i, 0)),
            pl.BlockSpec((MBLK, H), lambda i: (i, 0)),
        ],
        out_shape=[
            jax.ShapeDtypeStruct((N, H), jnp.float32),
            jax.ShapeDtypeStruct((N, H), jnp.float32),
        ],
    )(x, w1ab, b1r)


def _gather_body(pp, qq, srcv, dstv, stab_hbm, pre_out, efs_out,
                 sidx, didx, pbuf, qbuf, sobuf, stab, sem1, sem2):
    c = lax.axis_index("c")
    s = lax.axis_index("s")
    wid = s * NC + c
    nj = jnp.where(wid < NCHUNK % NW, NCHUNK // NW + 1, NCHUNK // NW)

    pltpu.sync_copy(stab_hbm, stab)

    def chunk(j, carry):
        off = (wid + j * NW) * GC
        pltpu.sync_copy(srcv.at[pl.ds(off, GC)], sidx)
        pltpu.sync_copy(dstv.at[pl.ds(off, GC)], didx)
        cp1 = pltpu.async_copy(pp.at[sidx], pbuf, sem1)
        cp2 = pltpu.async_copy(qq.at[didx], qbuf, sem2)
        cp1.wait()
        cp2.wait()

        def row(i, carry2):
            for k in range(8):
                sl = pl.ds(k * 16, 16)
                pbuf[i, sl] = pbuf[i, sl] + qbuf[i, sl]
            return carry2

        lax.fori_loop(0, GC, row, 0)

        for g in range(GC // 16):
            gsl = pl.ds(g * 16, 16)
            rs = sidx[gsl] * 4
            rd = didx[gsl] * 4
            for k in range(4):
                sobuf[k, gsl] = plsc.load_gather(stab, [rs + k])
                sobuf[4 + k, gsl] = plsc.load_gather(stab, [rd + k])

        pltpu.sync_copy(pbuf, pre_out.at[pl.ds(off, GC)])
        pltpu.sync_copy(sobuf, efs_out.at[:, pl.ds(off, GC)])
        return carry

    lax.fori_loop(0, nj, chunk, 0)


def _gather(p, q, src, dst, stab1d):
    mesh = plsc.VectorSubcoreMesh(core_axis_name="c", subcore_axis_name="s")
    return pl.kernel(
        _gather_body,
        out_type=[
            jax.ShapeDtypeStruct((E, H), jnp.float32),
            jax.ShapeDtypeStruct((8, E), jnp.float32),
        ],
        mesh=mesh,
        scratch_types=[
            pltpu.VMEM((GC,), jnp.int32),
            pltpu.VMEM((GC,), jnp.int32),
            pltpu.VMEM((GC, H), jnp.float32),
            pltpu.VMEM((GC, H), jnp.float32),
            pltpu.VMEM((8, GC), jnp.float32),
            pltpu.VMEM((4 * N,), jnp.float32),
            pltpu.SemaphoreType.DMA,
            pltpu.SemaphoreType.DMA,
        ],
        compiler_params=pltpu.CompilerParams(needs_layout_passes=False),
    )(p, q, src, dst, stab1d)


def _edge_body(pre_ref, efs_ref, ea_ref, w1call_ref, w2_ref, cw1_ref,
               vec_ref, h_ref, st_ref):
    nrow = pre_ref.shape[0]
    i8a = lax.broadcasted_iota(jnp.int32, (8, 8), 0)
    i8b = lax.broadcasted_iota(jnp.int32, (8, 8), 1)
    eye8 = (i8a == i8b).astype(jnp.float32)
    eye4 = eye8[0:4, 0:4]

    efs = efs_ref[...]
    r0 = efs[4:5, :] - efs[0:1, :]
    r1 = efs[5:6, :] - efs[1:2, :]
    r2 = efs[6:7, :] - efs[2:3, :]
    dist = jnp.sqrt(r0 * r0 + r1 * r1 + r2 * r2 + 1e-8)
    clp = jnp.maximum(dist, 1e-6)
    inv = 1.0 / clp

    eaT = lax.dot_general(eye4, ea_ref[...], (((1,), (1,)), ((), ())),
                          preferred_element_type=jnp.float32)

    def leg_mean_abs(a):
        co = jnp.cos(a)
        p2 = (3.0 * co * co - 1.0) * 0.5
        p3 = (5.0 * co * p2 - 2.0 * co) / 3.0
        return (1.0 + jnp.abs(co) + jnp.abs(p2) + jnp.abs(p3)) * 0.25

    a_s = leg_mean_abs(eaT[0:1, :]) * eaT[2:3, :]
    d_s = leg_mean_abs(eaT[1:2, :]) * eaT[3:4, :]
    gate = jnp.clip(1.0 + 0.6 * (a_s + d_s), 0.35, 2.5)

    theta = clp * (math.pi / CUTOFF)
    s1 = jnp.sin(theta)
    two_c = 2.0 * jnp.cos(theta)
    ginv = gate * inv
    rows = []
    prev = jnp.zeros_like(s1)
    cur = s1
    for _ in range(NR):
        rows.append(cur * ginv)
        prev, cur = cur, two_c * cur - prev
    rows.append(dist * (gate / CUTOFF))
    rows.append(efs[3:4, :] * efs[7:8, :] * gate)
    rows.append(jnp.abs(efs[3:4, :] - efs[7:8, :]) * gate)
    rows.append(jnp.zeros((5, nrow), jnp.float32))
    sm_t = jnp.concatenate(rows, axis=0)

    sp = lax.dot_general(sm_t, w1call_ref[...], (((0,), (0,)), ((), ())),
                         preferred_element_type=jnp.float32)

    h = _silu(pre_ref[...] + sp)
    m = jnp.mean(h, axis=-1, keepdims=True)
    hc = h - m
    var = jnp.mean(hc * hc, axis=-1, keepdims=True)
    hn = hc * lax.rsqrt(var + 1e-5) * vec_ref[0:1, :] + vec_ref[1:2, :]
    h2 = _silu(jnp.dot(hn, w2_ref[...], preferred_element_type=jnp.float32)
               + vec_ref[2:3, :])

    t = _silu(jnp.dot(h2, cw1_ref[...], preferred_element_type=jnp.float32)
              + vec_ref[3:4, 0:64])
    coord = jnp.sum(t * vec_ref[4:5, 0:64], axis=-1, keepdims=True) \
        + vec_ref[5:6, 0:1]

    g8 = jnp.concatenate(
        [r0 * inv, r1 * inv, r2 * inv, gate,
         jnp.zeros((4, nrow), jnp.float32)], axis=0)
    t8 = lax.dot_general(g8, eye8, (((0,), (0,)), ((), ())),
                         preferred_element_type=jnp.float32)
    ucg = t8[:, 0:3] * (coord * t8[:, 3:4])

    h_ref[...] = h2
    m8 = jnp.concatenate([ucg, jnp.zeros((nrow, 5), jnp.float32)], axis=1)
    st_ref[...] = lax.dot_general(eye8, m8, (((1,), (1,)), ((), ())),
                                  preferred_element_type=jnp.float32)


def _edge_mlp(pre, efs, edge_attr, w1call, w2, cw1, vecc):
    return pl.pallas_call(
        _edge_body,
        grid=(E // EBLK,),
        in_specs=[
            pl.BlockSpec((EBLK, H), lambda i: (i, 0)),
            pl.BlockSpec((8, EBLK), lambda i: (0, i)),
            pl.BlockSpec((EBLK, 4), lambda i: (i, 0)),
            pl.BlockSpec((24, H), lambda i: (0, 0)),
            pl.BlockSpec((H, H), lambda i: (0, 0)),
            pl.BlockSpec((H, 64), lambda i: (0, 0)),
            pl.BlockSpec((8, H), lambda i: (0, 0)),
        ],
        out_specs=[
            pl.BlockSpec((EBLK, H), lambda i: (i, 0)),
            pl.BlockSpec((8, EBLK), lambda i: (0, i)),
        ],
        out_shape=[
            jax.ShapeDtypeStruct((E, H), jnp.float32),
            jax.ShapeDtypeStruct((8, E), jnp.float32),
        ],
    )(pre, efs, edge_attr, w1call, w2, cw1, vecc)


HALF = NPAD // 2
ZPT = HALF // NS


def _scatter_body(hpay, spay, srcv, zrow, agg_out, smallp_out,
                  hbuf, spbuf, idxv, idx2, acc, slab, sem):
    c = lax.axis_index("c")
    s = lax.axis_index("s")
    nj = jnp.where(s < NCHUNK % NS, NCHUNK // NS + 1, NCHUNK // NS)

    pltpu.sync_copy(zrow, slab.at[pl.ds(s * ZPT, ZPT)])

    @pl.when(c == 0)
    def _():
        def zloop(i, carry):
            acc[pl.ds(i * 16, 16)] = jnp.zeros((16,), jnp.float32)
            return carry

        lax.fori_loop(0, (4 * NPAD) // 16, zloop, 0)

    plsc.subcore_barrier()

    ones16 = jnp.ones((16,), jnp.float32)
    base_node = c * HALF

    def chunk(j, carry):
        off = (s + j * NS) * GC
        pltpu.sync_copy(srcv.at[pl.ds(off, GC)], idxv)
        pltpu.sync_copy(hpay.at[pl.ds(off, GC)], hbuf)
        for g in range(GC // 16):
            gsl = pl.ds(g * 16, 16)
            r = idxv[gsl] - base_node
            ok = (r >= 0) & (r < HALF)
            idx2[gsl] = jnp.where(ok, r, HALF)
        pltpu.sync_copy(hbuf, slab.at[idx2], add=True)

        @pl.when(c == 0)
        def _():
            pltpu.sync_copy(spay.at[:, pl.ds(off, GC)], spbuf)
            for g in range(GC // 16):
                gsl = pl.ds(g * 16, 16)
                rows = idxv[gsl]
                for k in range(3):
                    plsc.addupdate_scatter(acc, [rows + k * NPAD],
                                           spbuf[k, gsl])
                plsc.addupdate_scatter(acc, [rows + 3 * NPAD], ones16)

        return carry

    lax.fori_loop(0, nj, chunk, 0)
    plsc.subcore_barrier()
    pltpu.sync_copy(slab.at[pl.ds(s * ZPT, ZPT)],
                    agg_out.at[pl.ds(c * HALF + s * ZPT, ZPT)])

    @pl.when(c == 0)
    def _():
        pltpu.sync_copy(acc, smallp_out.at[s])


def _scatter(hpay, spay, src, zrow):
    mesh = plsc.VectorSubcoreMesh(core_axis_name="c", subcore_axis_name="s")
    return pl.kernel(
        _scatter_body,
        out_type=[
            jax.ShapeDtypeStruct((NPAD, H), jnp.float32),
            jax.ShapeDtypeStruct((NS, 4 * NPAD), jnp.float32),
        ],
        mesh=mesh,
        scratch_types=[
            pltpu.VMEM((GC, H), jnp.float32),
            pltpu.VMEM((8, GC), jnp.float32),
            pltpu.VMEM((GC,), jnp.int32),
            pltpu.VMEM((GC,), jnp.int32),
            pltpu.VMEM((4 * NPAD,), jnp.float32),
            pltpu.VMEM_SHARED((HALF + 8, H), jnp.float32),
            pltpu.SemaphoreType.DMA,
        ],
        compiler_params=pltpu.CompilerParams(needs_layout_passes=False),
    )(hpay, spay, src, zrow)


def _final_body(x_ref, a_ref, sp_ref, pos_ref, nwa_ref, nwb_ref, vec_ref,
                xo_ref, po_ref):
    nrow = x_ref.shape[0]
    i4a = lax.broadcasted_iota(jnp.int32, (4, 4), 0)
    i4b = lax.broadcasted_iota(jnp.int32, (4, 4), 1)
    eye4 = (i4a == i4b).astype(jnp.float32)

    s4 = jnp.sum(sp_ref[...], axis=0)
    s4t = lax.dot_general(s4, eye4, (((0,), (0,)), ((), ())),
                          preferred_element_type=jnp.float32)
    cnt = jnp.maximum(s4t[:, 3:4], 1.0)
    delta = s4t[:, 0:3] / cnt
    agg = a_ref[...] / cnt

    xv = x_ref[...]
    t = _silu(jnp.dot(xv, nwa_ref[...], preferred_element_type=jnp.float32)
              + jnp.dot(agg, nwb_ref[...], preferred_element_type=jnp.float32)
              + vec_ref[0:1, :])
    m = jnp.mean(t, axis=-1, keepdims=True)
    tc = t - m
    var = jnp.mean(tc * tc, axis=-1, keepdims=True)
    tn = tc * lax.rsqrt(var + 1e-5) * vec_ref[1:2, :] + vec_ref[2:3, :]
    xo_ref[...] = xv + tn
    po_ref[:, 0:3] = pos_ref[...] + 0.1 * delta
    po_ref[:, 3:8] = jnp.zeros((nrow, 5), jnp.float32)


def _final(x, aggp, smallp3, pos, nw1a, nw1b, vece):
    return pl.pallas_call(
        _final_body,
        grid=(NPAD // FBLK,),
        in_specs=[
            pl.BlockSpec((FBLK, H), lambda i: (i, 0)),
            pl.BlockSpec((FBLK, H), lambda i: (i, 0)),
            pl.BlockSpec((NS, 4, FBLK), lambda i: (0, 0, i)),
            pl.BlockSpec((FBLK, 3), lambda i: (i, 0)),
            pl.BlockSpec((H, H), lambda i: (0, 0)),
            pl.BlockSpec((H, H), lambda i: (0, 0)),
            pl.BlockSpec((8, H), lambda i: (0, 0)),
        ],
        out_specs=[
            pl.BlockSpec((FBLK, H), lambda i: (i, 0)),
            pl.BlockSpec((FBLK, 8), lambda i: (i, 0)),
        ],
        out_shape=[
            jax.ShapeDtypeStruct((N, H), jnp.float32),
            jax.ShapeDtypeStruct((N, 8), jnp.float32),
        ],
    )(x, aggp, smallp3, pos, nw1a, nw1b, vece)


def kernel(x, pos, charge, edge_index, edge_attr, W1, b1, ln1_g, ln1_b, W2,
           b2, nW1, nb1, nln_g, nln_b, cW1, cb1, cW2, cb2):
    f32 = jnp.float32
    src = edge_index[0].astype(jnp.int32)
    dst = edge_index[1].astype(jnp.int32)

    w1ab = jnp.concatenate([W1[0:H], W1[H:2 * H]], axis=1)
    b1r = b1.reshape(1, H)
    p, q = _nodepre(x, w1ab, b1r)

    stab1d = jnp.concatenate([pos, charge], axis=1).reshape(-1)
    pre, efs = _gather(p, q, src, dst, stab1d)

    w1call = jnp.concatenate([W1[2 * H:], jnp.zeros((5, H), f32)], axis=0)
    vecc = jnp.stack([
        ln1_g, ln1_b, b2,
        jnp.pad(cb1, (0, H - 64)),
        jnp.pad(cW2[:, 0], (0, H - 64)),
        jnp.pad(cb2, (0, H - 1)),
        jnp.zeros((H,), f32), jnp.zeros((H,), f32),
    ])
    hpay, spay = _edge_mlp(pre, efs, edge_attr, w1call, W2, cW1, vecc)

    zrow = jnp.zeros((ZPT, H), f32)
    aggs, smallp = _scatter(hpay, spay, src, zrow)

    vece = jnp.stack([nb1, nln_g, nln_b] + [jnp.zeros((H,), f32)] * 5)
    xo, po = _final(x, aggs, smallp.reshape(NS, 4, NPAD), pos,
                    nW1[0:H], nW1[H:2 * H], vece)
    return (xo, po[:, 0:3])


# speedup vs baseline: 7.3742x; 1.2742x over previous
import math

import jax
import jax.numpy as jnp
from jax import lax
from jax.experimental import pallas as pl
from jax.experimental.pallas import tpu as pltpu
from jax.experimental.pallas import tpu_sc as plsc

N = 10000
E = 320000
H = 128
NR = 16
CUTOFF = 5.0

NC = 2
NS = 16
NW = NC * NS
GC = 128
NCHUNK = E // GC
NHALF = 2
EH = E // NHALF
HCHUNK = NCHUNK // NHALF
NPAD = 10240
NPT = NPAD // NS

MBLK = 400
FBLK = 512
EBLK = 640


def _silu(v):
    return v * jax.nn.sigmoid(v)


def _nodepre_body(x_ref, w_ref, b_ref, p_ref, q_ref):
    r = jnp.dot(x_ref[...], w_ref[...], preferred_element_type=jnp.float32)
    p_ref[...] = r[:, 0:128] + b_ref[...]
    q_ref[...] = r[:, 128:256]


def _nodepre(x, w1ab, b1r):
    return pl.pallas_call(
        _nodepre_body,
        grid=(N // MBLK,),
        in_specs=[
            pl.BlockSpec((MBLK, H), lambda i: (i, 0)),
            pl.BlockSpec((H, 2 * H), lambda i: (0, 0)),
            pl.BlockSpec((1, H), lambda i: (0, 0)),
        ],
        out_specs=[
            pl.BlockSpec((MBLK, H), lambda i: (i, 0)),
            pl.BlockSpec((MBLK, H), lambda i: (i, 0)),
        ],
        out_shape=[
            jax.ShapeDtypeStruct((N, H), jnp.float32),
            jax.ShapeDtypeStruct((N, H), jnp.float32),
        ],
    )(x, w1ab, b1r)


def _gather_body(base_chunk, pp, qq, srcv, dstv, stab_hbm, pre_out, efs_out,
                 sidx, didx, pbuf, qbuf, sobuf, stab, sem1, sem2):
    c = lax.axis_index("c")
    s = lax.axis_index("s")
    wid = s * NC + c
    nj = jnp.where(wid < HCHUNK % NW, HCHUNK // NW + 1, HCHUNK // NW)

    pltpu.sync_copy(stab_hbm, stab)

    def chunk(j, carry):
        off = (wid + j * NW) * GC
        goff = off + base_chunk * GC
        pltpu.sync_copy(srcv.at[pl.ds(goff, GC)], sidx)
        pltpu.sync_copy(dstv.at[pl.ds(goff, GC)], didx)
        cp1 = pltpu.async_copy(pp.at[sidx], pbuf, sem1)
        cp2 = pltpu.async_copy(qq.at[didx], qbuf, sem2)
        cp1.wait()
        cp2.wait()

        def row(i, carry2):
            for k in range(8):
                sl = pl.ds(k * 16, 16)
                pbuf[i, sl] = pbuf[i, sl] + qbuf[i, sl]
            return carry2

        lax.fori_loop(0, GC, row, 0)

        for g in range(GC // 16):
            gsl = pl.ds(g * 16, 16)
            rs = sidx[gsl] * 4
            rd = didx[gsl] * 4
            for k in range(4):
                sobuf[k, gsl] = plsc.load_gather(stab, [rs + k])
                sobuf[4 + k, gsl] = plsc.load_gather(stab, [rd + k])

        pltpu.sync_copy(pbuf, pre_out.at[pl.ds(off, GC)])
        pltpu.sync_copy(sobuf, efs_out.at[:, pl.ds(off, GC)])
        return carry

    lax.fori_loop(0, nj, chunk, 0)


def _gather(p, q, src, dst, stab1d, base_chunk):
    import functools
    mesh = plsc.VectorSubcoreMesh(core_axis_name="c", subcore_axis_name="s")
    return pl.kernel(
        functools.partial(_gather_body, base_chunk),
        out_type=[
            jax.ShapeDtypeStruct((EH, H), jnp.float32),
            jax.ShapeDtypeStruct((8, EH), jnp.float32),
        ],
        mesh=mesh,
        scratch_types=[
            pltpu.VMEM((GC,), jnp.int32),
            pltpu.VMEM((GC,), jnp.int32),
            pltpu.VMEM((GC, H), jnp.float32),
            pltpu.VMEM((GC, H), jnp.float32),
            pltpu.VMEM((8, GC), jnp.float32),
            pltpu.VMEM((4 * N,), jnp.float32),
            pltpu.SemaphoreType.DMA,
            pltpu.SemaphoreType.DMA,
        ],
        compiler_params=pltpu.CompilerParams(needs_layout_passes=False),
    )(p, q, src, dst, stab1d)


def _edge_body(pre_ref, efs_ref, ea_ref, w1call_ref, w2_ref, cw1_ref,
               vec_ref, h_ref, st_ref):
    nrow = pre_ref.shape[0]
    i8a = lax.broadcasted_iota(jnp.int32, (8, 8), 0)
    i8b = lax.broadcasted_iota(jnp.int32, (8, 8), 1)
    eye8 = (i8a == i8b).astype(jnp.float32)
    eye4 = eye8[0:4, 0:4]

    efs = efs_ref[...]
    r0 = efs[4:5, :] - efs[0:1, :]
    r1 = efs[5:6, :] - efs[1:2, :]
    r2 = efs[6:7, :] - efs[2:3, :]
    dist = jnp.sqrt(r0 * r0 + r1 * r1 + r2 * r2 + 1e-8)
    clp = jnp.maximum(dist, 1e-6)
    inv = 1.0 / clp

    eaT = lax.dot_general(eye4, ea_ref[...], (((1,), (1,)), ((), ())),
                          preferred_element_type=jnp.float32)

    def leg_mean_abs(a):
        co = jnp.cos(a)
        p2 = (3.0 * co * co - 1.0) * 0.5
        p3 = (5.0 * co * p2 - 2.0 * co) / 3.0
        return (1.0 + jnp.abs(co) + jnp.abs(p2) + jnp.abs(p3)) * 0.25

    a_s = leg_mean_abs(eaT[0:1, :]) * eaT[2:3, :]
    d_s = leg_mean_abs(eaT[1:2, :]) * eaT[3:4, :]
    gate = jnp.clip(1.0 + 0.6 * (a_s + d_s), 0.35, 2.5)

    theta = clp * (math.pi / CUTOFF)
    s1 = jnp.sin(theta)
    two_c = 2.0 * jnp.cos(theta)
    ginv = gate * inv
    rows = []
    prev = jnp.zeros_like(s1)
    cur = s1
    for _ in range(NR):
        rows.append(cur * ginv)
        prev, cur = cur, two_c * cur - prev
    rows.append(dist * (gate / CUTOFF))
    rows.append(efs[3:4, :] * efs[7:8, :] * gate)
    rows.append(jnp.abs(efs[3:4, :] - efs[7:8, :]) * gate)
    rows.append(jnp.zeros((5, nrow), jnp.float32))
    sm_t = jnp.concatenate(rows, axis=0)

    sp = lax.dot_general(sm_t, w1call_ref[...], (((0,), (0,)), ((), ())),
                         preferred_element_type=jnp.float32)

    h = _silu(pre_ref[...] + sp)
    m = jnp.mean(h, axis=-1, keepdims=True)
    hc = h - m
    var = jnp.mean(hc * hc, axis=-1, keepdims=True)
    hn = hc * lax.rsqrt(var + 1e-5) * vec_ref[0:1, :] + vec_ref[1:2, :]
    h2 = _silu(jnp.dot(hn, w2_ref[...], preferred_element_type=jnp.float32)
               + vec_ref[2:3, :])

    t = _silu(jnp.dot(h2, cw1_ref[...], preferred_element_type=jnp.float32)
              + vec_ref[3:4, 0:64])
    coord = jnp.sum(t * vec_ref[4:5, 0:64], axis=-1, keepdims=True) \
        + vec_ref[5:6, 0:1]

    g8 = jnp.concatenate(
        [r0 * inv, r1 * inv, r2 * inv, gate,
         jnp.zeros((4, nrow), jnp.float32)], axis=0)
    t8 = lax.dot_general(g8, eye8, (((0,), (0,)), ((), ())),
                         preferred_element_type=jnp.float32)
    ucg = t8[:, 0:3] * (coord * t8[:, 3:4])

    h_ref[...] = h2
    m8 = jnp.concatenate([ucg, jnp.zeros((nrow, 5), jnp.float32)], axis=1)
    st_ref[...] = lax.dot_general(eye8, m8, (((1,), (1,)), ((), ())),
                                  preferred_element_type=jnp.float32)


def _edge_mlp(pre, efs, edge_attr, w1call, w2, cw1, vecc, half):
    base = half * (EH // EBLK)
    return pl.pallas_call(
        _edge_body,
        grid=(EH // EBLK,),
        in_specs=[
            pl.BlockSpec((EBLK, H), lambda i: (i, 0)),
            pl.BlockSpec((8, EBLK), lambda i: (0, i)),
            pl.BlockSpec((EBLK, 4), lambda i: (i + base, 0)),
            pl.BlockSpec((24, H), lambda i: (0, 0)),
            pl.BlockSpec((H, H), lambda i: (0, 0)),
            pl.BlockSpec((H, 64), lambda i: (0, 0)),
            pl.BlockSpec((8, H), lambda i: (0, 0)),
        ],
        out_specs=[
            pl.BlockSpec((EBLK, H), lambda i: (i, 0)),
            pl.BlockSpec((8, EBLK), lambda i: (0, i)),
        ],
        out_shape=[
            jax.ShapeDtypeStruct((EH, H), jnp.float32),
            jax.ShapeDtypeStruct((8, EH), jnp.float32),
        ],
    )(pre, efs, edge_attr, w1call, w2, cw1, vecc)


HALF = NPAD // 2
ZPT = HALF // NS


def _scatter_body(base_chunk, hpay, spay, srcv, zrow, agg_out, smallp_out,
                  hbuf, spbuf, idxv, idx2, acc, slab, sem):
    c = lax.axis_index("c")
    s = lax.axis_index("s")
    nj = jnp.where(s < HCHUNK % NS, HCHUNK // NS + 1, HCHUNK // NS)

    pltpu.sync_copy(zrow, slab.at[pl.ds(s * ZPT, ZPT)])

    @pl.when(c == 0)
    def _():
        def zloop(i, carry):
            acc[pl.ds(i * 16, 16)] = jnp.zeros((16,), jnp.float32)
            return carry

        lax.fori_loop(0, (4 * NPAD) // 16, zloop, 0)

    plsc.subcore_barrier()

    ones16 = jnp.ones((16,), jnp.float32)
    base_node = c * HALF

    def chunk(j, carry):
        off = (s + j * NS) * GC
        goff = off + base_chunk * GC
        pltpu.sync_copy(srcv.at[pl.ds(goff, GC)], idxv)
        pltpu.sync_copy(hpay.at[pl.ds(off, GC)], hbuf)
        for g in range(GC // 16):
            gsl = pl.ds(g * 16, 16)
            r = idxv[gsl] - base_node
            ok = (r >= 0) & (r < HALF)
            idx2[gsl] = jnp.where(ok, r, HALF)
        pltpu.sync_copy(hbuf, slab.at[idx2], add=True)

        @pl.when(c == 0)
        def _():
            pltpu.sync_copy(spay.at[:, pl.ds(off, GC)], spbuf)
            for g in range(GC // 16):
                gsl = pl.ds(g * 16, 16)
                rows = idxv[gsl]
                for k in range(3):
                    plsc.addupdate_scatter(acc, [rows + k * NPAD],
                                           spbuf[k, gsl])
                plsc.addupdate_scatter(acc, [rows + 3 * NPAD], ones16)

        return carry

    lax.fori_loop(0, nj, chunk, 0)
    plsc.subcore_barrier()
    pltpu.sync_copy(slab.at[pl.ds(s * ZPT, ZPT)],
                    agg_out.at[pl.ds(c * HALF + s * ZPT, ZPT)])

    @pl.when(c == 0)
    def _():
        pltpu.sync_copy(acc, smallp_out.at[s])


def _scatter(hpay, spay, src, zrow, base_chunk):
    import functools
    mesh = plsc.VectorSubcoreMesh(core_axis_name="c", subcore_axis_name="s")
    return pl.kernel(
        functools.partial(_scatter_body, base_chunk),
        out_type=[
            jax.ShapeDtypeStruct((NPAD, H), jnp.float32),
            jax.ShapeDtypeStruct((NS, 4 * NPAD), jnp.float32),
        ],
        mesh=mesh,
        scratch_types=[
            pltpu.VMEM((GC, H), jnp.float32),
            pltpu.VMEM((8, GC), jnp.float32),
            pltpu.VMEM((GC,), jnp.int32),
            pltpu.VMEM((GC,), jnp.int32),
            pltpu.VMEM((4 * NPAD,), jnp.float32),
            pltpu.VMEM_SHARED((HALF + 8, H), jnp.float32),
            pltpu.SemaphoreType.DMA,
        ],
        compiler_params=pltpu.CompilerParams(needs_layout_passes=False),
    )(hpay, spay, src, zrow)


def _final_body(x_ref, a_ref, a2_ref, sp_ref, sp2_ref, pos_ref, nwa_ref,
                nwb_ref, vec_ref, xo_ref, po_ref):
    nrow = x_ref.shape[0]
    i4a = lax.broadcasted_iota(jnp.int32, (4, 4), 0)
    i4b = lax.broadcasted_iota(jnp.int32, (4, 4), 1)
    eye4 = (i4a == i4b).astype(jnp.float32)

    s4 = jnp.sum(sp_ref[...], axis=0) + jnp.sum(sp2_ref[...], axis=0)
    s4t = lax.dot_general(s4, eye4, (((0,), (0,)), ((), ())),
                          preferred_element_type=jnp.float32)
    cnt = jnp.maximum(s4t[:, 3:4], 1.0)
    delta = s4t[:, 0:3] / cnt
    agg = (a_ref[...] + a2_ref[...]) / cnt

    xv = x_ref[...]
    t = _silu(jnp.dot(xv, nwa_ref[...], preferred_element_type=jnp.float32)
              + jnp.dot(agg, nwb_ref[...], preferred_element_type=jnp.float32)
              + vec_ref[0:1, :])
    m = jnp.mean(t, axis=-1, keepdims=True)
    tc = t - m
    var = jnp.mean(tc * tc, axis=-1, keepdims=True)
    tn = tc * lax.rsqrt(var + 1e-5) * vec_ref[1:2, :] + vec_ref[2:3, :]
    xo_ref[...] = xv + tn
    po_ref[:, 0:3] = pos_ref[...] + 0.1 * delta
    po_ref[:, 3:8] = jnp.zeros((nrow, 5), jnp.float32)


def _final(x, aggp, aggp2, smallp3, smallp3b, pos, nw1a, nw1b, vece):
    return pl.pallas_call(
        _final_body,
        grid=(NPAD // FBLK,),
        in_specs=[
            pl.BlockSpec((FBLK, H), lambda i: (i, 0)),
            pl.BlockSpec((FBLK, H), lambda i: (i, 0)),
            pl.BlockSpec((FBLK, H), lambda i: (i, 0)),
            pl.BlockSpec((NS, 4, FBLK), lambda i: (0, 0, i)),
            pl.BlockSpec((NS, 4, FBLK), lambda i: (0, 0, i)),
            pl.BlockSpec((FBLK, 3), lambda i: (i, 0)),
            pl.BlockSpec((H, H), lambda i: (0, 0)),
            pl.BlockSpec((H, H), lambda i: (0, 0)),
            pl.BlockSpec((8, H), lambda i: (0, 0)),
        ],
        out_specs=[
            pl.BlockSpec((FBLK, H), lambda i: (i, 0)),
            pl.BlockSpec((FBLK, 8), lambda i: (i, 0)),
        ],
        out_shape=[
            jax.ShapeDtypeStruct((N, H), jnp.float32),
            jax.ShapeDtypeStruct((N, 8), jnp.float32),
        ],
    )(x, aggp, aggp2, smallp3, smallp3b, pos, nw1a, nw1b, vece)


def kernel(x, pos, charge, edge_index, edge_attr, W1, b1, ln1_g, ln1_b, W2,
           b2, nW1, nb1, nln_g, nln_b, cW1, cb1, cW2, cb2):
    f32 = jnp.float32
    src = edge_index[0].astype(jnp.int32)
    dst = edge_index[1].astype(jnp.int32)

    w1ab = jnp.concatenate([W1[0:H], W1[H:2 * H]], axis=1)
    b1r = b1.reshape(1, H)
    p, q = _nodepre(x, w1ab, b1r)

    stab1d = jnp.concatenate([pos, charge], axis=1).reshape(-1)
    pre1, efs1 = _gather(p, q, src, dst, stab1d, 0)
    pre2, efs2 = _gather(p, q, src, dst, stab1d, HCHUNK)

    w1call = jnp.concatenate([W1[2 * H:], jnp.zeros((5, H), f32)], axis=0)
    vecc = jnp.stack([
        ln1_g, ln1_b, b2,
        jnp.pad(cb1, (0, H - 64)),
        jnp.pad(cW2[:, 0], (0, H - 64)),
        jnp.pad(cb2, (0, H - 1)),
        jnp.zeros((H,), f32), jnp.zeros((H,), f32),
    ])
    hpay1, spay1 = _edge_mlp(pre1, efs1, edge_attr, w1call, W2, cW1, vecc, 0)
    hpay2, spay2 = _edge_mlp(pre2, efs2, edge_attr, w1call, W2, cW1, vecc, 1)

    zrow = jnp.zeros((ZPT, H), f32)
    aggs1, smallp1 = _scatter(hpay1, spay1, src, zrow, 0)
    aggs2, smallp2 = _scatter(hpay2, spay2, src, zrow, HCHUNK)

    vece = jnp.stack([nb1, nln_g, nln_b] + [jnp.zeros((H,), f32)] * 5)
    xo, po = _final(x, aggs1, aggs2, smallp1.reshape(NS, 4, NPAD),
                    smallp2.reshape(NS, 4, NPAD), pos,
                    nW1[0:H], nW1[H:2 * H], vece)
    return (xo, po[:, 0:3])


# speedup vs baseline: 8.5065x; 1.1536x over previous
import math

import jax
import jax.numpy as jnp
from jax import lax
from jax.experimental import pallas as pl
from jax.experimental.pallas import tpu as pltpu
from jax.experimental.pallas import tpu_sc as plsc

N = 10000
E = 320000
H = 128
NR = 16
CUTOFF = 5.0

NC = 2
NS = 16
NW = NC * NS
GC = 128
NCHUNK = E // GC
NHALF = 2
EH = E // NHALF
HCHUNK = NCHUNK // NHALF
NPAD = 10240
NJG = HCHUNK // NW
NJGMAX = NJG + 1
NJS = HCHUNK // NS
NJSMAX = NJS + 1
NPT = NPAD // NS

MBLK = 400
FBLK = 512
EBLK = 640


def _silu(v):
    return v * jax.nn.sigmoid(v)


def _nodepre_body(x_ref, w_ref, b_ref, p_ref, q_ref):
    r = jnp.dot(x_ref[...], w_ref[...], preferred_element_type=jnp.float32)
    p_ref[...] = r[:, 0:128] + b_ref[...]
    q_ref[...] = r[:, 128:256]


def _nodepre(x, w1ab, b1r):
    return pl.pallas_call(
        _nodepre_body,
        grid=(N // MBLK,),
        in_specs=[
            pl.BlockSpec((MBLK, H), lambda i: (i, 0)),
            pl.BlockSpec((H, 2 * H), lambda i: (0, 0)),
            pl.BlockSpec((1, H), lambda i: (0, 0)),
        ],
        out_specs=[
            pl.BlockSpec((MBLK, H), lambda i: (i, 0)),
            pl.BlockSpec((MBLK, H), lambda i: (i, 0)),
        ],
        out_shape=[
            jax.ShapeDtypeStruct((N, H), jnp.float32),
            jax.ShapeDtypeStruct((N, H), jnp.float32),
        ],
    )(x, w1ab, b1r)


def _gather_body(base_chunk, pp, qq, srcv, dstv, stab_hbm, pre_out, efs_out,
                 sidx_all, didx_all, pbuf0, qbuf0, pbuf1, qbuf1, sobuf,
                 stab, sa0, sb0, sa1, sb1):
    c = lax.axis_index("c")
    s = lax.axis_index("s")
    wid = s * NC + c
    rem = HCHUNK % NW
    nj = jnp.where(wid < rem, NJG + 1, NJG)
    cstart = wid * NJG + jnp.minimum(wid, rem)
    lstart = cstart * GC
    estart = base_chunk * GC + lstart

    pltpu.sync_copy(stab_hbm, stab)
    pltpu.sync_copy(srcv.at[pl.ds(estart, NJGMAX * GC)], sidx_all)
    pltpu.sync_copy(dstv.at[pl.ds(estart, NJGMAX * GC)], didx_all)

    bufs = ((pbuf0, qbuf0, sa0, sb0), (pbuf1, qbuf1, sa1, sb1))

    def start(j, b):
        pb, qb, sa, sb = bufs[b]
        isl = pl.ds(j * GC, GC)
        pltpu.async_copy(pp.at[sidx_all.at[isl]], pb, sa)
        pltpu.async_copy(qq.at[didx_all.at[isl]], qb, sb)

    def finish(j, b):
        pb, qb, sa, sb = bufs[b]
        isl = pl.ds(j * GC, GC)
        pltpu.make_async_copy(pp.at[sidx_all.at[isl]], pb, sa).wait()
        pltpu.make_async_copy(qq.at[didx_all.at[isl]], qb, sb).wait()

    def compute_write(j, b):
        pb, qb, _, _ = bufs[b]

        def row(i, carry2):
            for k in range(8):
                sl = pl.ds(k * 16, 16)
                pb[i, sl] = pb[i, sl] + qb[i, sl]
            return carry2

        lax.fori_loop(0, GC, row, 0)

        for g in range(GC // 16):
            gsl = pl.ds(g * 16, 16)
            rs = sidx_all[pl.ds(j * GC + g * 16, 16)] * 4
            rd = didx_all[pl.ds(j * GC + g * 16, 16)] * 4
            for k in range(4):
                sobuf[k, gsl] = plsc.load_gather(stab, [rs + k])
                sobuf[4 + k, gsl] = plsc.load_gather(stab, [rd + k])

        off = lstart + j * GC
        pltpu.sync_copy(pb, pre_out.at[pl.ds(off, GC)])
        pltpu.sync_copy(sobuf, efs_out.at[:, pl.ds(off, GC)])

    start(0, 0)

    def pair(jj, carry):
        for b in range(2):
            j = jj * 2 + b

            @pl.when(j < nj)
            def _():
                finish(j, b)

                @pl.when(j + 1 < nj)
                def _():
                    start(j + 1, 1 - b)

                compute_write(j, b)

        return carry

    lax.fori_loop(0, (NJGMAX + 1) // 2, pair, 0)


def _gather(p, q, src, dst, stab1d, base_chunk):
    import functools
    mesh = plsc.VectorSubcoreMesh(core_axis_name="c", subcore_axis_name="s")
    return pl.kernel(
        functools.partial(_gather_body, base_chunk),
        out_type=[
            jax.ShapeDtypeStruct((EH, H), jnp.float32),
            jax.ShapeDtypeStruct((8, EH), jnp.float32),
        ],
        mesh=mesh,
        scratch_types=[
            pltpu.VMEM((NJGMAX * GC,), jnp.int32),
            pltpu.VMEM((NJGMAX * GC,), jnp.int32),
            pltpu.VMEM((GC, H), jnp.float32),
            pltpu.VMEM((GC, H), jnp.float32),
            pltpu.VMEM((GC, H), jnp.float32),
            pltpu.VMEM((GC, H), jnp.float32),
            pltpu.VMEM((8, GC), jnp.float32),
            pltpu.VMEM((4 * N,), jnp.float32),
            pltpu.SemaphoreType.DMA,
            pltpu.SemaphoreType.DMA,
            pltpu.SemaphoreType.DMA,
            pltpu.SemaphoreType.DMA,
        ],
        compiler_params=pltpu.CompilerParams(needs_layout_passes=False),
    )(p, q, src, dst, stab1d)


def _edge_body(pre_ref, efs_ref, ea_ref, w1call_ref, w2_ref, cw1_ref,
               vec_ref, h_ref, st_ref):
    nrow = pre_ref.shape[0]
    i8a = lax.broadcasted_iota(jnp.int32, (8, 8), 0)
    i8b = lax.broadcasted_iota(jnp.int32, (8, 8), 1)
    eye8 = (i8a == i8b).astype(jnp.float32)
    eye4 = eye8[0:4, 0:4]

    efs = efs_ref[...]
    r0 = efs[4:5, :] - efs[0:1, :]
    r1 = efs[5:6, :] - efs[1:2, :]
    r2 = efs[6:7, :] - efs[2:3, :]
    dist = jnp.sqrt(r0 * r0 + r1 * r1 + r2 * r2 + 1e-8)
    clp = jnp.maximum(dist, 1e-6)
    inv = 1.0 / clp

    eaT = lax.dot_general(eye4, ea_ref[...], (((1,), (1,)), ((), ())),
                          preferred_element_type=jnp.float32)

    def leg_mean_abs(a):
        co = jnp.cos(a)
        p2 = (3.0 * co * co - 1.0) * 0.5
        p3 = (5.0 * co * p2 - 2.0 * co) / 3.0
        return (1.0 + jnp.abs(co) + jnp.abs(p2) + jnp.abs(p3)) * 0.25

    a_s = leg_mean_abs(eaT[0:1, :]) * eaT[2:3, :]
    d_s = leg_mean_abs(eaT[1:2, :]) * eaT[3:4, :]
    gate = jnp.clip(1.0 + 0.6 * (a_s + d_s), 0.35, 2.5)

    theta = clp * (math.pi / CUTOFF)
    s1 = jnp.sin(theta)
    two_c = 2.0 * jnp.cos(theta)
    ginv = gate * inv
    rows = []
    prev = jnp.zeros_like(s1)
    cur = s1
    for _ in range(NR):
        rows.append(cur * ginv)
        prev, cur = cur, two_c * cur - prev
    rows.append(dist * (gate / CUTOFF))
    rows.append(efs[3:4, :] * efs[7:8, :] * gate)
    rows.append(jnp.abs(efs[3:4, :] - efs[7:8, :]) * gate)
    rows.append(jnp.zeros((5, nrow), jnp.float32))
    sm_t = jnp.concatenate(rows, axis=0)

    sp = lax.dot_general(sm_t, w1call_ref[...], (((0,), (0,)), ((), ())),
                         preferred_element_type=jnp.float32)

    h = _silu(pre_ref[...] + sp)
    m = jnp.mean(h, axis=-1, keepdims=True)
    hc = h - m
    var = jnp.mean(hc * hc, axis=-1, keepdims=True)
    hn = hc * lax.rsqrt(var + 1e-5) * vec_ref[0:1, :] + vec_ref[1:2, :]
    h2 = _silu(jnp.dot(hn, w2_ref[...], preferred_element_type=jnp.float32)
               + vec_ref[2:3, :])

    t = _silu(jnp.dot(h2, cw1_ref[...], preferred_element_type=jnp.float32)
              + vec_ref[3:4, 0:64])
    coord = jnp.sum(t * vec_ref[4:5, 0:64], axis=-1, keepdims=True) \
        + vec_ref[5:6, 0:1]

    g8 = jnp.concatenate(
        [r0 * inv, r1 * inv, r2 * inv, gate,
         jnp.zeros((4, nrow), jnp.float32)], axis=0)
    t8 = lax.dot_general(g8, eye8, (((0,), (0,)), ((), ())),
                         preferred_element_type=jnp.float32)
    ucg = t8[:, 0:3] * (coord * t8[:, 3:4])

    h_ref[...] = h2
    m8 = jnp.concatenate([ucg, jnp.zeros((nrow, 5), jnp.float32)], axis=1)
    st_ref[...] = lax.dot_general(eye8, m8, (((1,), (1,)), ((), ())),
                                  preferred_element_type=jnp.float32)


def _edge_mlp(pre, efs, edge_attr, w1call, w2, cw1, vecc, half):
    base = half * (EH // EBLK)
    return pl.pallas_call(
        _edge_body,
        grid=(EH // EBLK,),
        in_specs=[
            pl.BlockSpec((EBLK, H), lambda i: (i, 0)),
            pl.BlockSpec((8, EBLK), lambda i: (0, i)),
            pl.BlockSpec((EBLK, 4), lambda i: (i + base, 0)),
            pl.BlockSpec((24, H), lambda i: (0, 0)),
            pl.BlockSpec((H, H), lambda i: (0, 0)),
            pl.BlockSpec((H, 64), lambda i: (0, 0)),
            pl.BlockSpec((8, H), lambda i: (0, 0)),
        ],
        out_specs=[
            pl.BlockSpec((EBLK, H), lambda i: (i, 0)),
            pl.BlockSpec((8, EBLK), lambda i: (0, i)),
        ],
        out_shape=[
            jax.ShapeDtypeStruct((EH, H), jnp.float32),
            jax.ShapeDtypeStruct((8, EH), jnp.float32),
        ],
    )(pre, efs, edge_attr, w1call, w2, cw1, vecc)


HALF = NPAD // 2
ZPT = HALF // NS


def _scatter_body(base_chunk, hpay, spay, srcv, zrow, agg_out, smallp_out,
                  hbuf0, spbuf0, hbuf1, spbuf1, idx_all, idx2, acc, slab,
                  h0, p0, h1, p1):
    c = lax.axis_index("c")
    s = lax.axis_index("s")
    rem = HCHUNK % NS
    nj = jnp.where(s < rem, NJS + 1, NJS)
    cstart = s * NJS + jnp.minimum(s, rem)
    lstart = cstart * GC
    estart = base_chunk * GC + lstart

    pltpu.sync_copy(srcv.at[pl.ds(estart, NJSMAX * GC)], idx_all)
    pltpu.sync_copy(zrow, slab.at[pl.ds(s * ZPT, ZPT)])

    @pl.when(c == 0)
    def _():
        def zloop(i, carry):
            acc[pl.ds(i * 16, 16)] = jnp.zeros((16,), jnp.float32)
            return carry

        lax.fori_loop(0, (4 * NPAD) // 16, zloop, 0)

    bufs = ((hbuf0, spbuf0, h0, p0), (hbuf1, spbuf1, h1, p1))

    def start(j, b):
        hb, spb, hs, ps = bufs[b]
        off = lstart + j * GC
        pltpu.async_copy(hpay.at[pl.ds(off, GC)], hb, hs)

        @pl.when(c == 0)
        def _():
            pltpu.async_copy(spay.at[:, pl.ds(off, GC)], spb, ps)

    def finish(j, b):
        hb, spb, hs, ps = bufs[b]
        off = lstart + j * GC
        pltpu.make_async_copy(hpay.at[pl.ds(off, GC)], hb, hs).wait()

        @pl.when(c == 0)
        def _():
            pltpu.make_async_copy(spay.at[:, pl.ds(off, GC)], spb, ps).wait()

    start(0, 0)
    plsc.subcore_barrier()

    ones16 = jnp.ones((16,), jnp.float32)
    base_node = c * HALF

    def work(j, b):
        hb, spb, _, _ = bufs[b]
        for g in range(GC // 16):
            gsl = pl.ds(g * 16, 16)
            r = idx_all[pl.ds(j * GC + g * 16, 16)] - base_node
            ok = (r >= 0) & (r < HALF)
            idx2[gsl] = jnp.where(ok, r, HALF)
        pltpu.sync_copy(hb, slab.at[idx2], add=True)

        @pl.when(c == 0)
        def _():
            for g in range(GC // 16):
                gsl = pl.ds(g * 16, 16)
                rows = idx_all[pl.ds(j * GC + g * 16, 16)]
                for k in range(3):
                    plsc.addupdate_scatter(acc, [rows + k * NPAD],
                                           spb[k, gsl])
                plsc.addupdate_scatter(acc, [rows + 3 * NPAD], ones16)

    def pair(jj, carry):
        for b in range(2):
            j = jj * 2 + b

            @pl.when(j < nj)
            def _():
                finish(j, b)

                @pl.when(j + 1 < nj)
                def _():
                    start(j + 1, 1 - b)

                work(j, b)

        return carry

    lax.fori_loop(0, (NJSMAX + 1) // 2, pair, 0)
    plsc.subcore_barrier()
    pltpu.sync_copy(slab.at[pl.ds(s * ZPT, ZPT)],
                    agg_out.at[pl.ds(c * HALF + s * ZPT, ZPT)])

    @pl.when(c == 0)
    def _():
        pltpu.sync_copy(acc, smallp_out.at[s])


def _scatter(hpay, spay, src, zrow, base_chunk):
    import functools
    mesh = plsc.VectorSubcoreMesh(core_axis_name="c", subcore_axis_name="s")
    return pl.kernel(
        functools.partial(_scatter_body, base_chunk),
        out_type=[
            jax.ShapeDtypeStruct((NPAD, H), jnp.float32),
            jax.ShapeDtypeStruct((NS, 4 * NPAD), jnp.float32),
        ],
        mesh=mesh,
        scratch_types=[
            pltpu.VMEM((GC, H), jnp.float32),
            pltpu.VMEM((8, GC), jnp.float32),
            pltpu.VMEM((GC, H), jnp.float32),
            pltpu.VMEM((8, GC), jnp.float32),
            pltpu.VMEM((NJSMAX * GC,), jnp.int32),
            pltpu.VMEM((GC,), jnp.int32),
            pltpu.VMEM((4 * NPAD,), jnp.float32),
            pltpu.VMEM_SHARED((HALF + 8, H), jnp.float32),
            pltpu.SemaphoreType.DMA,
            pltpu.SemaphoreType.DMA,
            pltpu.SemaphoreType.DMA,
            pltpu.SemaphoreType.DMA,
        ],
        compiler_params=pltpu.CompilerParams(needs_layout_passes=False),
    )(hpay, spay, src, zrow)


def _final_body(x_ref, a_ref, a2_ref, sp_ref, sp2_ref, pos_ref, nwa_ref,
                nwb_ref, vec_ref, xo_ref, po_ref):
    nrow = x_ref.shape[0]
    i4a = lax.broadcasted_iota(jnp.int32, (4, 4), 0)
    i4b = lax.broadcasted_iota(jnp.int32, (4, 4), 1)
    eye4 = (i4a == i4b).astype(jnp.float32)

    s4 = jnp.sum(sp_ref[...], axis=0) + jnp.sum(sp2_ref[...], axis=0)
    s4t = lax.dot_general(s4, eye4, (((0,), (0,)), ((), ())),
                          preferred_element_type=jnp.float32)
    cnt = jnp.maximum(s4t[:, 3:4], 1.0)
    delta = s4t[:, 0:3] / cnt
    agg = (a_ref[...] + a2_ref[...]) / cnt

    xv = x_ref[...]
    t = _silu(jnp.dot(xv, nwa_ref[...], preferred_element_type=jnp.float32)
              + jnp.dot(agg, nwb_ref[...], preferred_element_type=jnp.float32)
              + vec_ref[0:1, :])
    m = jnp.mean(t, axis=-1, keepdims=True)
    tc = t - m
    var = jnp.mean(tc * tc, axis=-1, keepdims=True)
    tn = tc * lax.rsqrt(var + 1e-5) * vec_ref[1:2, :] + vec_ref[2:3, :]
    xo_ref[...] = xv + tn
    po_ref[:, 0:3] = pos_ref[...] + 0.1 * delta
    po_ref[:, 3:8] = jnp.zeros((nrow, 5), jnp.float32)


def _final(x, aggp, aggp2, smallp3, smallp3b, pos, nw1a, nw1b, vece):
    return pl.pallas_call(
        _final_body,
        grid=(NPAD // FBLK,),
        in_specs=[
            pl.BlockSpec((FBLK, H), lambda i: (i, 0)),
            pl.BlockSpec((FBLK, H), lambda i: (i, 0)),
            pl.BlockSpec((FBLK, H), lambda i: (i, 0)),
            pl.BlockSpec((NS, 4, FBLK), lambda i: (0, 0, i)),
            pl.BlockSpec((NS, 4, FBLK), lambda i: (0, 0, i)),
            pl.BlockSpec((FBLK, 3), lambda i: (i, 0)),
            pl.BlockSpec((H, H), lambda i: (0, 0)),
            pl.BlockSpec((H, H), lambda i: (0, 0)),
            pl.BlockSpec((8, H), lambda i: (0, 0)),
        ],
        out_specs=[
            pl.BlockSpec((FBLK, H), lambda i: (i, 0)),
            pl.BlockSpec((FBLK, 8), lambda i: (i, 0)),
        ],
        out_shape=[
            jax.ShapeDtypeStruct((N, H), jnp.float32),
            jax.ShapeDtypeStruct((N, 8), jnp.float32),
        ],
    )(x, aggp, aggp2, smallp3, smallp3b, pos, nw1a, nw1b, vece)


def kernel(x, pos, charge, edge_index, edge_attr, W1, b1, ln1_g, ln1_b, W2,
           b2, nW1, nb1, nln_g, nln_b, cW1, cb1, cW2, cb2):
    f32 = jnp.float32
    zpad = jnp.zeros((GC,), jnp.int32)
    src = jnp.concatenate([edge_index[0].astype(jnp.int32), zpad])
    dst = jnp.concatenate([edge_index[1].astype(jnp.int32), zpad])

    w1ab = jnp.concatenate([W1[0:H], W1[H:2 * H]], axis=1)
    b1r = b1.reshape(1, H)
    p, q = _nodepre(x, w1ab, b1r)

    stab1d = jnp.concatenate([pos, charge], axis=1).reshape(-1)
    pre1, efs1 = _gather(p, q, src, dst, stab1d, 0)
    pre2, efs2 = _gather(p, q, src, dst, stab1d, HCHUNK)

    w1call = jnp.concatenate([W1[2 * H:], jnp.zeros((5, H), f32)], axis=0)
    vecc = jnp.stack([
        ln1_g, ln1_b, b2,
        jnp.pad(cb1, (0, H - 64)),
        jnp.pad(cW2[:, 0], (0, H - 64)),
        jnp.pad(cb2, (0, H - 1)),
        jnp.zeros((H,), f32), jnp.zeros((H,), f32),
    ])
    hpay1, spay1 = _edge_mlp(pre1, efs1, edge_attr, w1call, W2, cW1, vecc, 0)
    hpay2, spay2 = _edge_mlp(pre2, efs2, edge_attr, w1call, W2, cW1, vecc, 1)

    zrow = jnp.zeros((ZPT, H), f32)
    aggs1, smallp1 = _scatter(hpay1, spay1, src, zrow, 0)
    aggs2, smallp2 = _scatter(hpay2, spay2, src, zrow, HCHUNK)

    vece = jnp.stack([nb1, nln_g, nln_b] + [jnp.zeros((H,), f32)] * 5)
    xo, po = _final(x, aggs1, aggs2, smallp1.reshape(NS, 4, NPAD),
                    smallp2.reshape(NS, 4, NPAD), pos,
                    nW1[0:H], nW1[H:2 * H], vece)
    return (xo, po[:, 0:3])


# speedup vs baseline: 8.5582x; 1.0061x over previous
import math

import jax
import jax.numpy as jnp
from jax import lax
from jax.experimental import pallas as pl
from jax.experimental.pallas import tpu as pltpu
from jax.experimental.pallas import tpu_sc as plsc

N = 10000
E = 320000
H = 128
NR = 16
CUTOFF = 5.0

NC = 2
NS = 16
NW = NC * NS
GC = 128
NCHUNK = E // GC
NHALF = 2
EH = E // NHALF
HCHUNK = NCHUNK // NHALF
NPAD = 10240
NJG = HCHUNK // NW
NJGMAX = NJG + 1
NJS = HCHUNK // NS
NJSMAX = NJS + 1
NPT = NPAD // NS

MBLK = 400
FBLK = 512
EBLK = 640


def _silu(v):
    return v / (1.0 + jnp.exp(-v))


def _nodepre_body(x_ref, w_ref, b_ref, p_ref, q_ref):
    r = jnp.dot(x_ref[...], w_ref[...], preferred_element_type=jnp.float32)
    p_ref[...] = r[:, 0:128] + b_ref[...]
    q_ref[...] = r[:, 128:256]


def _nodepre(x, w1ab, b1r):
    return pl.pallas_call(
        _nodepre_body,
        grid=(N // MBLK,),
        in_specs=[
            pl.BlockSpec((MBLK, H), lambda i: (i, 0)),
            pl.BlockSpec((H, 2 * H), lambda i: (0, 0)),
            pl.BlockSpec((1, H), lambda i: (0, 0)),
        ],
        out_specs=[
            pl.BlockSpec((MBLK, H), lambda i: (i, 0)),
            pl.BlockSpec((MBLK, H), lambda i: (i, 0)),
        ],
        out_shape=[
            jax.ShapeDtypeStruct((N, H), jnp.float32),
            jax.ShapeDtypeStruct((N, H), jnp.float32),
        ],
    )(x, w1ab, b1r)


def _gather_body(base_chunk, pp, qq, srcv, dstv, stab_hbm, pre_out, efs_out,
                 sidx_all, didx_all, pbuf0, qbuf0, pbuf1, qbuf1, sobuf,
                 stab, sa0, sb0, sa1, sb1):
    c = lax.axis_index("c")
    s = lax.axis_index("s")
    wid = s * NC + c
    rem = HCHUNK % NW
    nj = jnp.where(wid < rem, NJG + 1, NJG)
    cstart = wid * NJG + jnp.minimum(wid, rem)
    lstart = cstart * GC
    estart = base_chunk * GC + lstart

    pltpu.sync_copy(stab_hbm, stab)
    pltpu.sync_copy(srcv.at[pl.ds(estart, NJGMAX * GC)], sidx_all)
    pltpu.sync_copy(dstv.at[pl.ds(estart, NJGMAX * GC)], didx_all)

    bufs = ((pbuf0, qbuf0, sa0, sb0), (pbuf1, qbuf1, sa1, sb1))

    def start(j, b):
        pb, qb, sa, sb = bufs[b]
        isl = pl.ds(j * GC, GC)
        pltpu.async_copy(pp.at[sidx_all.at[isl]], pb, sa)
        pltpu.async_copy(qq.at[didx_all.at[isl]], qb, sb)

    def finish(j, b):
        pb, qb, sa, sb = bufs[b]
        isl = pl.ds(j * GC, GC)
        pltpu.make_async_copy(pp.at[sidx_all.at[isl]], pb, sa).wait()
        pltpu.make_async_copy(qq.at[didx_all.at[isl]], qb, sb).wait()

    def compute_write(j, b):
        pb, qb, _, _ = bufs[b]

        def row(i, carry2):
            for k in range(8):
                sl = pl.ds(k * 16, 16)
                pb[i, sl] = pb[i, sl] + qb[i, sl]
            return carry2

        lax.fori_loop(0, GC, row, 0)

        for g in range(GC // 16):
            gsl = pl.ds(g * 16, 16)
            rs = sidx_all[pl.ds(j * GC + g * 16, 16)] * 4
            rd = didx_all[pl.ds(j * GC + g * 16, 16)] * 4
            for k in range(4):
                sobuf[k, gsl] = plsc.load_gather(stab, [rs + k])
                sobuf[4 + k, gsl] = plsc.load_gather(stab, [rd + k])

        off = lstart + j * GC
        pltpu.sync_copy(pb, pre_out.at[pl.ds(off, GC)])
        pltpu.sync_copy(sobuf, efs_out.at[:, pl.ds(off, GC)])

    start(0, 0)

    def pair(jj, carry):
        for b in range(2):
            j = jj * 2 + b

            @pl.when(j < nj)
            def _():
                finish(j, b)

                @pl.when(j + 1 < nj)
                def _():
                    start(j + 1, 1 - b)

                compute_write(j, b)

        return carry

    lax.fori_loop(0, (NJGMAX + 1) // 2, pair, 0)


def _gather(p, q, src, dst, stab1d, base_chunk):
    import functools
    mesh = plsc.VectorSubcoreMesh(core_axis_name="c", subcore_axis_name="s")
    return pl.kernel(
        functools.partial(_gather_body, base_chunk),
        out_type=[
            jax.ShapeDtypeStruct((EH, H), jnp.float32),
            jax.ShapeDtypeStruct((8, EH), jnp.float32),
        ],
        mesh=mesh,
        scratch_types=[
            pltpu.VMEM((NJGMAX * GC,), jnp.int32),
            pltpu.VMEM((NJGMAX * GC,), jnp.int32),
            pltpu.VMEM((GC, H), jnp.float32),
            pltpu.VMEM((GC, H), jnp.float32),
            pltpu.VMEM((GC, H), jnp.float32),
            pltpu.VMEM((GC, H), jnp.float32),
            pltpu.VMEM((8, GC), jnp.float32),
            pltpu.VMEM((4 * N,), jnp.float32),
            pltpu.SemaphoreType.DMA,
            pltpu.SemaphoreType.DMA,
            pltpu.SemaphoreType.DMA,
            pltpu.SemaphoreType.DMA,
        ],
        compiler_params=pltpu.CompilerParams(needs_layout_passes=False),
    )(p, q, src, dst, stab1d)


def _edge_body(pre_ref, efs_ref, ea_ref, w1call_ref, w2_ref, cw1_ref,
               vec_ref, h_ref, st_ref):
    nrow = pre_ref.shape[0]
    i8a = lax.broadcasted_iota(jnp.int32, (8, 8), 0)
    i8b = lax.broadcasted_iota(jnp.int32, (8, 8), 1)
    eye8 = (i8a == i8b).astype(jnp.float32)
    eye4 = eye8[0:4, 0:4]

    efs = efs_ref[...]
    r0 = efs[4:5, :] - efs[0:1, :]
    r1 = efs[5:6, :] - efs[1:2, :]
    r2 = efs[6:7, :] - efs[2:3, :]
    dist = jnp.sqrt(r0 * r0 + r1 * r1 + r2 * r2 + 1e-8)
    clp = jnp.maximum(dist, 1e-6)
    inv = 1.0 / clp

    eaT = lax.dot_general(eye4, ea_ref[...], (((1,), (1,)), ((), ())),
                          preferred_element_type=jnp.float32)

    def leg_mean_abs(a):
        co = jnp.cos(a)
        p2 = (3.0 * co * co - 1.0) * 0.5
        p3 = (5.0 * co * p2 - 2.0 * co) / 3.0
        return (1.0 + jnp.abs(co) + jnp.abs(p2) + jnp.abs(p3)) * 0.25

    a_s = leg_mean_abs(eaT[0:1, :]) * eaT[2:3, :]
    d_s = leg_mean_abs(eaT[1:2, :]) * eaT[3:4, :]
    gate = jnp.clip(1.0 + 0.6 * (a_s + d_s), 0.35, 2.5)

    theta = clp * (math.pi / CUTOFF)
    s1 = jnp.sin(theta)
    two_c = 2.0 * jnp.cos(theta)
    ginv = gate * inv
    rows = []
    prev = jnp.zeros_like(s1)
    cur = s1
    for _ in range(NR):
        rows.append(cur * ginv)
        prev, cur = cur, two_c * cur - prev
    rows.append(dist * (gate / CUTOFF))
    rows.append(efs[3:4, :] * efs[7:8, :] * gate)
    rows.append(jnp.abs(efs[3:4, :] - efs[7:8, :]) * gate)
    rows.append(jnp.zeros((5, nrow), jnp.float32))
    sm_t = jnp.concatenate(rows, axis=0)

    sp = lax.dot_general(sm_t, w1call_ref[...], (((0,), (0,)), ((), ())),
                         preferred_element_type=jnp.float32)

    h = _silu(pre_ref[...] + sp)
    m = jnp.mean(h, axis=-1, keepdims=True)
    hc = h - m
    var = jnp.mean(hc * hc, axis=-1, keepdims=True)
    hn = hc * lax.rsqrt(var + 1e-5) * vec_ref[0:1, :] + vec_ref[1:2, :]
    h2 = _silu(jnp.dot(hn, w2_ref[...], preferred_element_type=jnp.float32)
               + vec_ref[2:3, :])

    t = _silu(jnp.dot(h2, cw1_ref[...], preferred_element_type=jnp.float32)
              + vec_ref[3:4, 0:64])
    coord = jnp.sum(t * vec_ref[4:5, 0:64], axis=-1, keepdims=True) \
        + vec_ref[5:6, 0:1]

    g8 = jnp.concatenate(
        [r0 * inv, r1 * inv, r2 * inv, gate,
         jnp.zeros((4, nrow), jnp.float32)], axis=0)
    t8 = lax.dot_general(g8, eye8, (((0,), (0,)), ((), ())),
                         preferred_element_type=jnp.float32)
    ucg = t8[:, 0:3] * (coord * t8[:, 3:4])

    h_ref[...] = h2
    m8 = jnp.concatenate([ucg, jnp.zeros((nrow, 5), jnp.float32)], axis=1)
    st_ref[...] = lax.dot_general(eye8, m8, (((1,), (1,)), ((), ())),
                                  preferred_element_type=jnp.float32)


def _edge_mlp(pre, efs, edge_attr, w1call, w2, cw1, vecc, half):
    base = half * (EH // EBLK)
    return pl.pallas_call(
        _edge_body,
        grid=(EH // EBLK,),
        in_specs=[
            pl.BlockSpec((EBLK, H), lambda i: (i, 0)),
            pl.BlockSpec((8, EBLK), lambda i: (0, i)),
            pl.BlockSpec((EBLK, 4), lambda i: (i + base, 0)),
            pl.BlockSpec((24, H), lambda i: (0, 0)),
            pl.BlockSpec((H, H), lambda i: (0, 0)),
            pl.BlockSpec((H, 64), lambda i: (0, 0)),
            pl.BlockSpec((8, H), lambda i: (0, 0)),
        ],
        out_specs=[
            pl.BlockSpec((EBLK, H), lambda i: (i, 0)),
            pl.BlockSpec((8, EBLK), lambda i: (0, i)),
        ],
        out_shape=[
            jax.ShapeDtypeStruct((EH, H), jnp.float32),
            jax.ShapeDtypeStruct((8, EH), jnp.float32),
        ],
    )(pre, efs, edge_attr, w1call, w2, cw1, vecc)


HALF = NPAD // 2
ZPT = HALF // NS


def _scatter_body(base_chunk, hpay, spay, srcv, zrow, agg_out, smallp_out,
                  hbuf0, spbuf0, hbuf1, spbuf1, idx_all, idx2, acc, slab,
                  h0, p0, h1, p1):
    c = lax.axis_index("c")
    s = lax.axis_index("s")
    rem = HCHUNK % NS
    nj = jnp.where(s < rem, NJS + 1, NJS)
    cstart = s * NJS + jnp.minimum(s, rem)
    lstart = cstart * GC
    estart = base_chunk * GC + lstart

    pltpu.sync_copy(srcv.at[pl.ds(estart, NJSMAX * GC)], idx_all)
    pltpu.sync_copy(zrow, slab.at[pl.ds(s * ZPT, ZPT)])

    @pl.when(c == 0)
    def _():
        def zloop(i, carry):
            acc[pl.ds(i * 16, 16)] = jnp.zeros((16,), jnp.float32)
            return carry

        lax.fori_loop(0, (4 * NPAD) // 16, zloop, 0)

    bufs = ((hbuf0, spbuf0, h0, p0), (hbuf1, spbuf1, h1, p1))

    def start(j, b):
        hb, spb, hs, ps = bufs[b]
        off = lstart + j * GC
        pltpu.async_copy(hpay.at[pl.ds(off, GC)], hb, hs)

        @pl.when(c == 0)
        def _():
            pltpu.async_copy(spay.at[:, pl.ds(off, GC)], spb, ps)

    def finish(j, b):
        hb, spb, hs, ps = bufs[b]
        off = lstart + j * GC
        pltpu.make_async_copy(hpay.at[pl.ds(off, GC)], hb, hs).wait()

        @pl.when(c == 0)
        def _():
            pltpu.make_async_copy(spay.at[:, pl.ds(off, GC)], spb, ps).wait()

    start(0, 0)
    plsc.subcore_barrier()

    ones16 = jnp.ones((16,), jnp.float32)
    base_node = c * HALF

    def work(j, b):
        hb, spb, _, _ = bufs[b]
        for g in range(GC // 16):
            gsl = pl.ds(g * 16, 16)
            r = idx_all[pl.ds(j * GC + g * 16, 16)] - base_node
            ok = (r >= 0) & (r < HALF)
            idx2[gsl] = jnp.where(ok, r, HALF)
        pltpu.sync_copy(hb, slab.at[idx2], add=True)

        @pl.when(c == 0)
        def _():
            for g in range(GC // 16):
                gsl = pl.ds(g * 16, 16)
                rows = idx_all[pl.ds(j * GC + g * 16, 16)]
                for k in range(3):
                    plsc.addupdate_scatter(acc, [rows + k * NPAD],
                                           spb[k, gsl])
                plsc.addupdate_scatter(acc, [rows + 3 * NPAD], ones16)

    def pair(jj, carry):
        for b in range(2):
            j = jj * 2 + b

            @pl.when(j < nj)
            def _():
                finish(j, b)

                @pl.when(j + 1 < nj)
                def _():
                    start(j + 1, 1 - b)

                work(j, b)

        return carry

    lax.fori_loop(0, (NJSMAX + 1) // 2, pair, 0)
    plsc.subcore_barrier()
    pltpu.sync_copy(slab.at[pl.ds(s * ZPT, ZPT)],
                    agg_out.at[pl.ds(c * HALF + s * ZPT, ZPT)])

    @pl.when(c == 0)
    def _():
        pltpu.sync_copy(acc, smallp_out.at[s])


def _scatter(hpay, spay, src, zrow, base_chunk):
    import functools
    mesh = plsc.VectorSubcoreMesh(core_axis_name="c", subcore_axis_name="s")
    return pl.kernel(
        functools.partial(_scatter_body, base_chunk),
        out_type=[
            jax.ShapeDtypeStruct((NPAD, H), jnp.float32),
            jax.ShapeDtypeStruct((NS, 4 * NPAD), jnp.float32),
        ],
        mesh=mesh,
        scratch_types=[
            pltpu.VMEM((GC, H), jnp.float32),
            pltpu.VMEM((8, GC), jnp.float32),
            pltpu.VMEM((GC, H), jnp.float32),
            pltpu.VMEM((8, GC), jnp.float32),
            pltpu.VMEM((NJSMAX * GC,), jnp.int32),
            pltpu.VMEM((GC,), jnp.int32),
            pltpu.VMEM((4 * NPAD,), jnp.float32),
            pltpu.VMEM_SHARED((HALF + 8, H), jnp.float32),
            pltpu.SemaphoreType.DMA,
            pltpu.SemaphoreType.DMA,
            pltpu.SemaphoreType.DMA,
            pltpu.SemaphoreType.DMA,
        ],
        compiler_params=pltpu.CompilerParams(needs_layout_passes=False),
    )(hpay, spay, src, zrow)


def _final_body(x_ref, a_ref, a2_ref, sp_ref, sp2_ref, pos_ref, nwa_ref,
                nwb_ref, vec_ref, xo_ref, po_ref):
    nrow = x_ref.shape[0]
    i4a = lax.broadcasted_iota(jnp.int32, (4, 4), 0)
    i4b = lax.broadcasted_iota(jnp.int32, (4, 4), 1)
    eye4 = (i4a == i4b).astype(jnp.float32)

    s4 = jnp.sum(sp_ref[...], axis=0) + jnp.sum(sp2_ref[...], axis=0)
    s4t = lax.dot_general(s4, eye4, (((0,), (0,)), ((), ())),
                          preferred_element_type=jnp.float32)
    cnt = jnp.maximum(s4t[:, 3:4], 1.0)
    delta = s4t[:, 0:3] / cnt
    agg = (a_ref[...] + a2_ref[...]) / cnt

    xv = x_ref[...]
    t = _silu(jnp.dot(xv, nwa_ref[...], preferred_element_type=jnp.float32)
              + jnp.dot(agg, nwb_ref[...], preferred_element_type=jnp.float32)
              + vec_ref[0:1, :])
    m = jnp.mean(t, axis=-1, keepdims=True)
    tc = t - m
    var = jnp.mean(tc * tc, axis=-1, keepdims=True)
    tn = tc * lax.rsqrt(var + 1e-5) * vec_ref[1:2, :] + vec_ref[2:3, :]
    xo_ref[...] = xv + tn
    po_ref[:, 0:3] = pos_ref[...] + 0.1 * delta
    po_ref[:, 3:8] = jnp.zeros((nrow, 5), jnp.float32)


def _final(x, aggp, aggp2, smallp3, smallp3b, pos, nw1a, nw1b, vece):
    return pl.pallas_call(
        _final_body,
        grid=(NPAD // FBLK,),
        in_specs=[
            pl.BlockSpec((FBLK, H), lambda i: (i, 0)),
            pl.BlockSpec((FBLK, H), lambda i: (i, 0)),
            pl.BlockSpec((FBLK, H), lambda i: (i, 0)),
            pl.BlockSpec((NS, 4, FBLK), lambda i: (0, 0, i)),
            pl.BlockSpec((NS, 4, FBLK), lambda i: (0, 0, i)),
            pl.BlockSpec((FBLK, 3), lambda i: (i, 0)),
            pl.BlockSpec((H, H), lambda i: (0, 0)),
            pl.BlockSpec((H, H), lambda i: (0, 0)),
            pl.BlockSpec((8, H), lambda i: (0, 0)),
        ],
        out_specs=[
            pl.BlockSpec((FBLK, H), lambda i: (i, 0)),
            pl.BlockSpec((FBLK, 8), lambda i: (i, 0)),
        ],
        out_shape=[
            jax.ShapeDtypeStruct((N, H), jnp.float32),
            jax.ShapeDtypeStruct((N, 8), jnp.float32),
        ],
    )(x, aggp, aggp2, smallp3, smallp3b, pos, nw1a, nw1b, vece)


def kernel(x, pos, charge, edge_index, edge_attr, W1, b1, ln1_g, ln1_b, W2,
           b2, nW1, nb1, nln_g, nln_b, cW1, cb1, cW2, cb2):
    f32 = jnp.float32
    zpad = jnp.zeros((GC,), jnp.int32)
    src = jnp.concatenate([edge_index[0].astype(jnp.int32), zpad])
    dst = jnp.concatenate([edge_index[1].astype(jnp.int32), zpad])

    w1ab = jnp.concatenate([W1[0:H], W1[H:2 * H]], axis=1)
    b1r = b1.reshape(1, H)
    p, q = _nodepre(x, w1ab, b1r)

    stab1d = jnp.concatenate([pos, charge], axis=1).reshape(-1)
    pre1, efs1 = _gather(p, q, src, dst, stab1d, 0)
    pre2, efs2 = _gather(p, q, src, dst, stab1d, HCHUNK)

    w1call = jnp.concatenate([W1[2 * H:], jnp.zeros((5, H), f32)], axis=0)
    vecc = jnp.stack([
        ln1_g, ln1_b, b2,
        jnp.pad(cb1, (0, H - 64)),
        jnp.pad(cW2[:, 0], (0, H - 64)),
        jnp.pad(cb2, (0, H - 1)),
        jnp.zeros((H,), f32), jnp.zeros((H,), f32),
    ])
    hpay1, spay1 = _edge_mlp(pre1, efs1, edge_attr, w1call, W2, cW1, vecc, 0)
    hpay2, spay2 = _edge_mlp(pre2, efs2, edge_attr, w1call, W2, cW1, vecc, 1)

    zrow = jnp.zeros((ZPT, H), f32)
    aggs1, smallp1 = _scatter(hpay1, spay1, src, zrow, 0)
    aggs2, smallp2 = _scatter(hpay2, spay2, src, zrow, HCHUNK)

    vece = jnp.stack([nb1, nln_g, nln_b] + [jnp.zeros((H,), f32)] * 5)
    xo, po = _final(x, aggs1, aggs2, smallp1.reshape(NS, 4, NPAD),
                    smallp2.reshape(NS, 4, NPAD), pos,
                    nW1[0:H], nW1[H:2 * H], vece)
    return (xo, po[:, 0:3])
